# baseline (speedup 1.0000x reference)
import functools

import jax
import jax.numpy as jnp
from jax.experimental import pallas as pl

_N, _D = 16384, 50
_P = _D // 2
_ROWS = 8192
_KA = 128


def _body(xa_ref, W1_ref, W2_ref, b2_ref, W3_ref, bc_ref,
          rW1_ref, rb1_ref, rW2_ref, rb2_ref, rW3_ref, rb3_ref,
          mu_ref, sig_ref):
    W2 = W2_ref[:]
    nb2 = b2_ref[:]
    xt = xa_ref[:]
    s2 = jnp.zeros((_ROWS, 128), jnp.float32)
    for j in range(_P):
        W1j = W1_ref[_KA * j:_KA * (j + 1), :]
        p1 = jnp.dot(xt, W1j, preferred_element_type=jnp.float32)
        g = jnp.maximum(p1.astype(jnp.bfloat16), jnp.bfloat16(0.0))
        p2 = jnp.dot(g, W2, preferred_element_type=jnp.float32)
        s2 = s2 + jnp.maximum(p2, nb2)

    cnt = jnp.sum(xt[:, _D:2 * _D].astype(jnp.float32), axis=1,
                  keepdims=True)
    pooled = (jnp.dot(s2, W3_ref[:], preferred_element_type=jnp.float32)
              + cnt * bc_ref[:])
    r = jnp.maximum(
        jnp.dot(pooled, rW1_ref[:], preferred_element_type=jnp.float32)
        + rb1_ref[:], 0.0)
    r = jnp.maximum(
        jnp.dot(r, rW2_ref[:], preferred_element_type=jnp.float32)
        + rb2_ref[:], 0.0)
    g = (jnp.dot(r, rW3_ref[:], preferred_element_type=jnp.float32)
         + rb3_ref[:])
    mu_ref[:] = g[:, :64]
    sig_ref[:] = jnp.logaddexp(g[:, 64:], 0.0)


@functools.partial(jax.jit, static_argnames=("interpret",))
def _run(xa, W1s, W2b, b2b, W3s, bc, rW1, rb1c, rW2, rb2, rW3, rb3,
         interpret=False):
    grid = (_N // _ROWS,)

    def rep(shape):
        return pl.BlockSpec(shape, lambda i: tuple(0 for _ in shape))

    mu, sig = pl.pallas_call(
        _body,
        grid=grid,
        in_specs=[
            pl.BlockSpec((_ROWS, _KA), lambda i: (i, 0)),
            rep((_KA * _P, 128)), rep((128, 128)), rep((1, 128)),
            rep((128, 64)), rep((1, 64)),
            rep((64, 64)), rep((1, 64)),
            rep((64, 64)), rep((1, 64)),
            rep((64, 128)), rep((1, 128)),
        ],
        out_specs=[pl.BlockSpec((_ROWS, 64), lambda i: (i, 0)),
                   pl.BlockSpec((_ROWS, 64), lambda i: (i, 0))],
        out_shape=[
            jax.ShapeDtypeStruct((_N, 64), jnp.float32),
            jax.ShapeDtypeStruct((_N, 64), jnp.float32),
        ],
        interpret=interpret,
    )(xa, W1s, W2b, b2b, W3s, bc, rW1, rb1c, rW2, rb2, rW3, rb3)
    return mu, sig


def kernel(x, mask, hW1, hb1, hW2, hb2, hW3, hb3,
           rW1, rb1, rW2, rb2, rW3, rb3):
    maskf = mask.astype(jnp.float32)
    xa = jnp.concatenate([x * maskf, maskf], axis=1)
    xa = jnp.pad(xa, ((0, 0), (0, _KA - 2 * _D))).astype(jnp.bfloat16)

    dim_ids = jnp.arange(_D, dtype=jnp.float32)[:, None]
    B = dim_ids * hW1[1:2, :] + hb1[None, :]
    w0 = hW1[0, :]

    z64 = jnp.zeros((64,), jnp.float32)
    zP64 = jnp.zeros((_P, 64), jnp.float32)
    row_xL = jnp.concatenate([w0, z64])
    row_xR = jnp.concatenate([z64, w0])
    row_mL = jnp.concatenate([B[0::2], zP64], axis=1)
    row_mR = jnp.concatenate([zP64, B[1::2]], axis=1)
    r_iota = jnp.arange(_KA)[None, :, None]
    base = 2 * jnp.arange(_P)[:, None, None]
    W1s = ((r_iota == base) * row_xL[None, None, :]
           + (r_iota == base + 1) * row_xR[None, None, :]
           + (r_iota == base + _D) * row_mL[:, None, :]
           + (r_iota == base + _D + 1) * row_mR[:, None, :])
    W1s = W1s.reshape(_P * _KA, _KA).astype(jnp.bfloat16)

    z = jnp.zeros((64, 64), jnp.float32)
    W2b = jnp.block([[hW2, z], [z, hW2]]).astype(jnp.bfloat16)
    b2b = -jnp.concatenate([hb2, hb2])[None, :]

    W3s = jnp.concatenate([hW3, hW3], axis=0)

    q = jax.nn.relu(hb2) @ hW3
    bc = (hb3 + q)[None, :]
    rb1c = (rb1 + _D * ((hb2 @ hW3 - q) @ rW1))[None, :]

    return _run(xa, W1s, W2b, b2b, W3s, bc,
                rW1, rb1c, rW2, rb2[None, :], rW3, rb3[None, :])


# speedup vs baseline: 1.2078x; 1.2078x over previous
import functools

import jax
import jax.numpy as jnp
from jax.experimental import pallas as pl

_N, _D = 16384, 50
_P = _D // 2
_ROWS = 2048
_KA = 128


def _body(xa_ref, W1_ref, W2_ref, b2_ref, W3_ref, bc_ref,
          rW1_ref, rb1_ref, rW2_ref, rb2_ref, rW3_ref, rb3_ref,
          mu_ref, sig_ref):
    W2 = W2_ref[:]
    nb2 = b2_ref[:]
    xt = xa_ref[:]
    s2 = jnp.zeros((_ROWS, 128), jnp.float32)
    for j in range(_P):
        W1j = W1_ref[_KA * j:_KA * (j + 1), :]
        p1 = jnp.dot(xt, W1j, preferred_element_type=jnp.float32)
        g = jnp.maximum(p1.astype(jnp.bfloat16), jnp.bfloat16(0.0))
        p2 = jnp.dot(g, W2, preferred_element_type=jnp.float32)
        s2 = s2 + jnp.maximum(p2, nb2)

    cnt = jnp.sum(xt[:, _D:2 * _D].astype(jnp.float32), axis=1,
                  keepdims=True)
    pooled = (jnp.dot(s2, W3_ref[:], preferred_element_type=jnp.float32)
              + cnt * bc_ref[:])
    r = jnp.maximum(
        jnp.dot(pooled, rW1_ref[:], preferred_element_type=jnp.float32)
        + rb1_ref[:], 0.0)
    r = jnp.maximum(
        jnp.dot(r, rW2_ref[:], preferred_element_type=jnp.float32)
        + rb2_ref[:], 0.0)
    g = (jnp.dot(r, rW3_ref[:], preferred_element_type=jnp.float32)
         + rb3_ref[:])
    mu_ref[:] = g[:, :64]
    sig_ref[:] = jnp.logaddexp(g[:, 64:], 0.0)


@functools.partial(jax.jit, static_argnames=("interpret",))
def _run(xa, W1s, W2b, b2b, W3s, bc, rW1, rb1c, rW2, rb2, rW3, rb3,
         interpret=False):
    grid = (_N // _ROWS,)

    def rep(shape):
        return pl.BlockSpec(shape, lambda i: tuple(0 for _ in shape))

    mu, sig = pl.pallas_call(
        _body,
        grid=grid,
        in_specs=[
            pl.BlockSpec((_ROWS, _KA), lambda i: (i, 0)),
            rep((_KA * _P, 128)), rep((128, 128)), rep((1, 128)),
            rep((128, 64)), rep((1, 64)),
            rep((64, 64)), rep((1, 64)),
            rep((64, 64)), rep((1, 64)),
            rep((64, 128)), rep((1, 128)),
        ],
        out_specs=[pl.BlockSpec((_ROWS, 64), lambda i: (i, 0)),
                   pl.BlockSpec((_ROWS, 64), lambda i: (i, 0))],
        out_shape=[
            jax.ShapeDtypeStruct((_N, 64), jnp.float32),
            jax.ShapeDtypeStruct((_N, 64), jnp.float32),
        ],
        interpret=interpret,
    )(xa, W1s, W2b, b2b, W3s, bc, rW1, rb1c, rW2, rb2, rW3, rb3)
    return mu, sig


def kernel(x, mask, hW1, hb1, hW2, hb2, hW3, hb3,
           rW1, rb1, rW2, rb2, rW3, rb3):
    maskf = mask.astype(jnp.float32)
    xa = jnp.concatenate([x * maskf, maskf], axis=1)
    xa = jnp.pad(xa, ((0, 0), (0, _KA - 2 * _D))).astype(jnp.bfloat16)

    dim_ids = jnp.arange(_D, dtype=jnp.float32)[:, None]
    B = dim_ids * hW1[1:2, :] + hb1[None, :]
    w0 = hW1[0, :]

    z64 = jnp.zeros((64,), jnp.float32)
    zP64 = jnp.zeros((_P, 64), jnp.float32)
    row_xL = jnp.concatenate([w0, z64])
    row_xR = jnp.concatenate([z64, w0])
    row_mL = jnp.concatenate([B[0::2], zP64], axis=1)
    row_mR = jnp.concatenate([zP64, B[1::2]], axis=1)
    r_iota = jnp.arange(_KA)[None, :, None]
    base = 2 * jnp.arange(_P)[:, None, None]
    W1s = ((r_iota == base) * row_xL[None, None, :]
           + (r_iota == base + 1) * row_xR[None, None, :]
           + (r_iota == base + _D) * row_mL[:, None, :]
           + (r_iota == base + _D + 1) * row_mR[:, None, :])
    W1s = W1s.reshape(_P * _KA, _KA).astype(jnp.bfloat16)

    z = jnp.zeros((64, 64), jnp.float32)
    W2b = jnp.block([[hW2, z], [z, hW2]]).astype(jnp.bfloat16)
    b2b = -jnp.concatenate([hb2, hb2])[None, :]

    W3s = jnp.concatenate([hW3, hW3], axis=0)

    q = jax.nn.relu(hb2) @ hW3
    bc = (hb3 + q)[None, :]
    rb1c = (rb1 + _D * ((hb2 @ hW3 - q) @ rW1))[None, :]

    return _run(xa, W1s, W2b, b2b, W3s, bc,
                rW1, rb1c, rW2, rb2[None, :], rW3, rb3[None, :])


# speedup vs baseline: 1.2142x; 1.0052x over previous
import functools

import jax
import jax.numpy as jnp
from jax.experimental import pallas as pl

_N, _D = 16384, 50
_P = _D // 2
_ROWS = 4096
_KA = 128


def _body(xa_ref, W1_ref, W2_ref, b2_ref, W3_ref, bc_ref,
          rW1_ref, rb1_ref, rW2_ref, rb2_ref, rW3_ref, rb3_ref,
          mu_ref, sig_ref):
    W2 = W2_ref[:]
    nb2 = b2_ref[:]
    xt = xa_ref[:]
    s2 = jnp.zeros((_ROWS, 128), jnp.float32)
    for g0 in range(0, _P, 5):
        terms = []
        for j in range(g0, g0 + 5):
            W1j = W1_ref[_KA * j:_KA * (j + 1), :]
            p1 = jnp.dot(xt, W1j, preferred_element_type=jnp.float32)
            g = jnp.maximum(p1.astype(jnp.bfloat16), jnp.bfloat16(0.0))
            p2 = jnp.dot(g, W2, preferred_element_type=jnp.float32)
            terms.append(jnp.maximum(p2, nb2))
        s2 = s2 + ((terms[0] + terms[1]) + (terms[2] + terms[3])
                   + terms[4])

    cnt = jnp.sum(xt[:, _D:2 * _D].astype(jnp.float32), axis=1,
                  keepdims=True)
    pooled = (jnp.dot(s2, W3_ref[:], preferred_element_type=jnp.float32)
              + cnt * bc_ref[:])
    r = jnp.maximum(
        jnp.dot(pooled, rW1_ref[:], preferred_element_type=jnp.float32)
        + rb1_ref[:], 0.0)
    r = jnp.maximum(
        jnp.dot(r, rW2_ref[:], preferred_element_type=jnp.float32)
        + rb2_ref[:], 0.0)
    g = (jnp.dot(r, rW3_ref[:], preferred_element_type=jnp.float32)
         + rb3_ref[:])
    mu_ref[:] = g[:, :64]
    sig_ref[:] = jnp.logaddexp(g[:, 64:], 0.0)


@functools.partial(jax.jit, static_argnames=("interpret",))
def _run(xa, W1s, W2b, b2b, W3s, bc, rW1, rb1c, rW2, rb2, rW3, rb3,
         interpret=False):
    grid = (_N // _ROWS,)

    def rep(shape):
        return pl.BlockSpec(shape, lambda i: tuple(0 for _ in shape))

    mu, sig = pl.pallas_call(
        _body,
        grid=grid,
        in_specs=[
            pl.BlockSpec((_ROWS, _KA), lambda i: (i, 0)),
            rep((_KA * _P, 128)), rep((128, 128)), rep((1, 128)),
            rep((128, 64)), rep((1, 64)),
            rep((64, 64)), rep((1, 64)),
            rep((64, 64)), rep((1, 64)),
            rep((64, 128)), rep((1, 128)),
        ],
        out_specs=[pl.BlockSpec((_ROWS, 64), lambda i: (i, 0)),
                   pl.BlockSpec((_ROWS, 64), lambda i: (i, 0))],
        out_shape=[
            jax.ShapeDtypeStruct((_N, 64), jnp.float32),
            jax.ShapeDtypeStruct((_N, 64), jnp.float32),
        ],
        interpret=interpret,
    )(xa, W1s, W2b, b2b, W3s, bc, rW1, rb1c, rW2, rb2, rW3, rb3)
    return mu, sig


def kernel(x, mask, hW1, hb1, hW2, hb2, hW3, hb3,
           rW1, rb1, rW2, rb2, rW3, rb3):
    maskf = mask.astype(jnp.float32)
    xa = jnp.concatenate([x * maskf, maskf], axis=1)
    xa = jnp.pad(xa, ((0, 0), (0, _KA - 2 * _D))).astype(jnp.bfloat16)

    dim_ids = jnp.arange(_D, dtype=jnp.float32)[:, None]
    B = dim_ids * hW1[1:2, :] + hb1[None, :]
    w0 = hW1[0, :]

    z64 = jnp.zeros((64,), jnp.float32)
    zP64 = jnp.zeros((_P, 64), jnp.float32)
    row_xL = jnp.concatenate([w0, z64])
    row_xR = jnp.concatenate([z64, w0])
    row_mL = jnp.concatenate([B[0::2], zP64], axis=1)
    row_mR = jnp.concatenate([zP64, B[1::2]], axis=1)
    r_iota = jnp.arange(_KA)[None, :, None]
    base = 2 * jnp.arange(_P)[:, None, None]
    W1s = ((r_iota == base) * row_xL[None, None, :]
           + (r_iota == base + 1) * row_xR[None, None, :]
           + (r_iota == base + _D) * row_mL[:, None, :]
           + (r_iota == base + _D + 1) * row_mR[:, None, :])
    W1s = W1s.reshape(_P * _KA, _KA).astype(jnp.bfloat16)

    z = jnp.zeros((64, 64), jnp.float32)
    W2b = jnp.block([[hW2, z], [z, hW2]]).astype(jnp.bfloat16)
    b2b = -jnp.concatenate([hb2, hb2])[None, :]

    W3s = jnp.concatenate([hW3, hW3], axis=0)

    q = jax.nn.relu(hb2) @ hW3
    bc = (hb3 + q)[None, :]
    rb1c = (rb1 + _D * ((hb2 @ hW3 - q) @ rW1))[None, :]

    return _run(xa, W1s, W2b, b2b, W3s, bc,
                rW1, rb1c, rW2, rb2[None, :], rW3, rb3[None, :])
